# baseline (speedup 1.0000x reference)
import functools

import jax
import jax.numpy as jnp
from jax import lax
from jax.experimental import pallas as pl
from jax.experimental.pallas import tpu as pltpu
from jax.experimental.pallas import tpu_sc as plsc

_STRIDE = 16.0
_POS_W = 2.0
_NEG_W = 1.0
_PAD_W = 0.2

_NC = 2
_NS = 16
_LN = 16


@functools.lru_cache(maxsize=None)
def _build_sc_call(B, L, FW):
    NW = _NC * _NS
    assert B % NW == 0 and (B // NW) % 2 == 0, (B, NW)
    RPW = B // NW
    mesh = plsc.VectorSubcoreMesh(
        core_axis_name="c", subcore_axis_name="s",
        num_cores=_NC, num_subcores=_NS)

    @functools.partial(
        pl.kernel,
        mesh=mesh,
        compiler_params=pltpu.CompilerParams(needs_layout_passes=False),
        out_type=(
            jax.ShapeDtypeStruct((B, FW), jnp.float32),
            jax.ShapeDtypeStruct((B, FW), jnp.float32),
            jax.ShapeDtypeStruct((B, 2, FW), jnp.float32),
        ),
        scratch_types=[
            pltpu.VMEM((2, L), jnp.float32),
            pltpu.VMEM((2, L), jnp.float32),
            pltpu.VMEM((RPW,), jnp.int32),
            pltpu.VMEM((L,), jnp.int32),
            pltpu.VMEM((L,), jnp.int32),
            pltpu.VMEM((FW,), jnp.float32),
            pltpu.VMEM((FW,), jnp.float32),
            pltpu.VMEM((FW,), jnp.float32),
            pltpu.VMEM((FW,), jnp.float32),
            pltpu.VMEM((2, FW), jnp.float32),
            pltpu.VMEM((2, FW), jnp.float32),
            pltpu.SemaphoreType.DMA,
            pltpu.SemaphoreType.DMA,
            pltpu.SemaphoreType.DMA,
            pltpu.SemaphoreType.DMA,
            pltpu.SemaphoreType.DMA,
            pltpu.SemaphoreType.DMA,
            pltpu.SemaphoreType.DMA,
            pltpu.SemaphoreType.DMA,
        ],
    )
    def sc_call(pos_hbm, rfw_hbm, cls_hbm, w_hbm, delta_hbm,
                pos_v0, pos_v1, rfw_v, ivl_v0, ivl_v1,
                cls_v0, cls_v1, w_v0, w_v1, dl_v0, dl_v1,
                sin0, sin1, sc0, sc1, sw0, sw1, sd0, sd1):
        pos_vs = (pos_v0, pos_v1)
        ivl_vs = (ivl_v0, ivl_v1)
        cls_vs = (cls_v0, cls_v1)
        w_vs = (w_v0, w_v1)
        dl_vs = (dl_v0, dl_v1)
        sem_in = (sin0, sin1)
        sem_cls = (sc0, sc1)
        sem_w = (sw0, sw1)
        sem_dl = (sd0, sd1)

        cid = lax.axis_index("c")
        sid = lax.axis_index("s")
        wid = sid * _NC + cid
        base = wid * RPW

        pltpu.make_async_copy(pos_hbm.at[base], pos_vs[0], sem_in[0]).start()
        pltpu.sync_copy(rfw_hbm.at[pl.ds(base, RPW)], rfw_v)

        iota = lax.iota(jnp.int32, _LN)
        iota_f = iota.astype(jnp.float32)
        zeros_i = jnp.zeros((_LN,), jnp.int32)
        ones_i = jnp.full((_LN,), 1, jnp.int32)
        zeros_f = jnp.zeros((_LN,), jnp.float32)
        ones_f = jnp.full((_LN,), 1.0, jnp.float32)
        pos_w = jnp.full((_LN,), _POS_W, jnp.float32)

        def pair_fn(rp, _):
            for p in (0, 1):
                r = 2 * rp + p
                b = base + r
                pos_v, cls_v, w_v, dl_v = pos_vs[p], cls_vs[p], w_vs[p], dl_vs[p]
                ivl_v = ivl_vs[p]

                @pl.when(r + 1 < RPW)
                def _prefetch():
                    pltpu.make_async_copy(
                        pos_hbm.at[b + 1], pos_vs[1 - p], sem_in[1 - p]).start()

                @pl.when(rp >= 1)
                def _drain():
                    pltpu.make_async_copy(cls_v, cls_hbm.at[b - 2], sem_cls[p]).wait()
                    pltpu.make_async_copy(w_v, w_hbm.at[b - 2], sem_w[p]).wait()
                    pltpu.make_async_copy(dl_v, delta_hbm.at[b - 2], sem_dl[p]).wait()

                rfw_b = plsc.load_gather(rfw_v, [jnp.full((_LN,), r, jnp.int32)])

                @pl.when(rp == 0)
                def _first_fill():
                    @plsc.parallel_loop(0, FW // _LN, unroll=8)
                    def _fill(c):
                        p0 = c * _LN
                        cls_v[pl.ds(p0, _LN)] = zeros_f
                        dval = jnp.float32(-0.5) - (p0 + iota_f)
                        dl_v[0, pl.ds(p0, _LN)] = dval
                        dl_v[1, pl.ds(p0, _LN)] = dval

                @pl.when(rp >= 1)
                def _undo():
                    @plsc.parallel_loop(0, L // _LN, unroll=4)
                    def _undo_loop(c):
                        l0 = c * _LN
                        old = ivl_v[pl.ds(l0, _LN)]
                        oldc = jnp.minimum(jnp.maximum(old, 0), FW - 1)
                        dval = jnp.float32(-0.5) - oldc.astype(jnp.float32)
                        plsc.store_scatter(cls_v, [oldc], zeros_f)
                        plsc.store_scatter(dl_v, [zeros_i, oldc], dval)
                        plsc.store_scatter(dl_v, [ones_i, oldc], dval)

                pltpu.make_async_copy(pos_hbm.at[b], pos_v, sem_in[p]).wait()

                @plsc.parallel_loop(0, L // _LN, unroll=4)
                def _pass_a(c):
                    l0 = c * _LN
                    x1 = pos_v[0, pl.ds(l0, _LN)]
                    x2 = pos_v[1, pl.ds(l0, _LN)]
                    cs = (x1 + x2) * jnp.float32(0.5 / _STRIDE)
                    ivl = jnp.where(cs < 0.0, -1, cs.astype(jnp.int32))
                    ivl_v[pl.ds(l0, _LN)] = ivl

                @plsc.parallel_loop(0, L // _LN, unroll=4)
                def _pass_b(c):
                    l0 = c * _LN
                    li = l0 + iota
                    ivl = ivl_v[pl.ds(l0, _LN)]
                    prev = plsc.load_gather(ivl_v, [jnp.maximum(li - 1, 0)])
                    valid = ((ivl != prev) | (li == 0)) & (ivl >= 0) & (ivl < FW)
                    x1 = pos_v[0, pl.ds(l0, _LN)]
                    x2 = pos_v[1, pl.ds(l0, _LN)]
                    fvl = ivl.astype(jnp.float32) + 0.5
                    d0 = x1 * jnp.float32(1.0 / _STRIDE) - fvl
                    d1 = x2 * jnp.float32(1.0 / _STRIDE) - fvl
                    plsc.store_scatter(cls_v, [ivl], ones_f, mask=valid)
                    plsc.store_scatter(dl_v, [zeros_i, ivl], d0, mask=valid)
                    plsc.store_scatter(dl_v, [ones_i, ivl], d1, mask=valid)

                @plsc.parallel_loop(0, FW // _LN, unroll=8)
                def _wfill(c):
                    p0 = c * _LN
                    fvec = p0 + iota
                    cls_c = cls_v[pl.ds(p0, _LN)]
                    wdef = jnp.where(fvec <= rfw_b, _NEG_W, _PAD_W)
                    w_v[pl.ds(p0, _LN)] = jnp.where(cls_c == 1.0, _POS_W, wdef)

                pltpu.make_async_copy(cls_v, cls_hbm.at[b], sem_cls[p]).start()
                pltpu.make_async_copy(w_v, w_hbm.at[b], sem_w[p]).start()
                pltpu.make_async_copy(dl_v, delta_hbm.at[b], sem_dl[p]).start()
            return _
        lax.fori_loop(0, RPW // 2, pair_fn, None)

        for p in (0, 1):
            b_last = base + RPW - 2 + p
            pltpu.make_async_copy(cls_vs[p], cls_hbm.at[b_last], sem_cls[p]).wait()
            pltpu.make_async_copy(w_vs[p], w_hbm.at[b_last], sem_w[p]).wait()
            pltpu.make_async_copy(dl_vs[p], delta_hbm.at[b_last], sem_dl[p]).wait()

    return sc_call


def kernel(split_line_pos, feat_width, real_features_width):
    B, L, _ = split_line_pos.shape
    FW = 8192
    del feat_width
    rfw = real_features_width.astype(jnp.int32)
    pos_planes = split_line_pos.transpose(0, 2, 1)
    cls, w, delta_planes = _build_sc_call(B, L, FW)(pos_planes, rfw)
    return cls, w, delta_planes.transpose(0, 2, 1)


# speedup vs baseline: 1.0064x; 1.0064x over previous
import functools

import jax
import jax.numpy as jnp
from jax import lax
from jax.experimental import pallas as pl
from jax.experimental.pallas import tpu as pltpu
from jax.experimental.pallas import tpu_sc as plsc

_STRIDE = 16.0
_POS_W = 2.0
_NEG_W = 1.0
_PAD_W = 0.2

_NC = 2
_NS = 16
_LN = 16


@functools.lru_cache(maxsize=None)
def _build_sc_call(B, L, FW):
    NW = _NC * _NS
    assert B % NW == 0 and (B // NW) % 2 == 0, (B, NW)
    RPW = B // NW
    mesh = plsc.VectorSubcoreMesh(
        core_axis_name="c", subcore_axis_name="s",
        num_cores=_NC, num_subcores=_NS)

    @functools.partial(
        pl.kernel,
        mesh=mesh,
        compiler_params=pltpu.CompilerParams(needs_layout_passes=False),
        out_type=(
            jax.ShapeDtypeStruct((B, FW), jnp.float32),
            jax.ShapeDtypeStruct((B, FW), jnp.float32),
            jax.ShapeDtypeStruct((B, 2, FW), jnp.float32),
        ),
        scratch_types=[
            pltpu.VMEM((2, L), jnp.float32),
            pltpu.VMEM((2, L), jnp.float32),
            pltpu.VMEM((RPW,), jnp.int32),
            pltpu.VMEM((L,), jnp.int32),
            pltpu.VMEM((L,), jnp.int32),
            pltpu.VMEM((FW,), jnp.float32),
            pltpu.VMEM((FW,), jnp.float32),
            pltpu.VMEM((FW,), jnp.float32),
            pltpu.VMEM((FW,), jnp.float32),
            pltpu.VMEM((2, FW), jnp.float32),
            pltpu.VMEM((2, FW), jnp.float32),
            pltpu.SemaphoreType.DMA,
            pltpu.SemaphoreType.DMA,
            pltpu.SemaphoreType.DMA,
            pltpu.SemaphoreType.DMA,
            pltpu.SemaphoreType.DMA,
            pltpu.SemaphoreType.DMA,
            pltpu.SemaphoreType.DMA,
            pltpu.SemaphoreType.DMA,
        ],
    )
    def sc_call(pos_hbm, rfw_hbm, cls_hbm, w_hbm, delta_hbm,
                pos_v0, pos_v1, rfw_v, ivl_v0, ivl_v1,
                cls_v0, cls_v1, w_v0, w_v1, dl_v0, dl_v1,
                sin0, sin1, sc0, sc1, sw0, sw1, sd0, sd1):
        pos_vs = (pos_v0, pos_v1)
        ivl_vs = (ivl_v0, ivl_v1)
        cls_vs = (cls_v0, cls_v1)
        w_vs = (w_v0, w_v1)
        dl_vs = (dl_v0, dl_v1)
        sem_in = (sin0, sin1)
        sem_cls = (sc0, sc1)
        sem_w = (sw0, sw1)
        sem_dl = (sd0, sd1)

        cid = lax.axis_index("c")
        sid = lax.axis_index("s")
        wid = sid * _NC + cid
        base = wid * RPW

        pltpu.make_async_copy(pos_hbm.at[base], pos_vs[0], sem_in[0]).start()
        pltpu.sync_copy(rfw_hbm.at[pl.ds(base, RPW)], rfw_v)

        iota = lax.iota(jnp.int32, _LN)
        iota_f = iota.astype(jnp.float32)
        zeros_i = jnp.zeros((_LN,), jnp.int32)
        ones_i = jnp.full((_LN,), 1, jnp.int32)
        zeros_f = jnp.zeros((_LN,), jnp.float32)
        ones_f = jnp.full((_LN,), 1.0, jnp.float32)
        pos_w = jnp.full((_LN,), _POS_W, jnp.float32)

        def pair_fn(rp, _):
            for p in (0, 1):
                r = 2 * rp + p
                b = base + r
                pos_v, cls_v, w_v, dl_v = pos_vs[p], cls_vs[p], w_vs[p], dl_vs[p]
                ivl_v = ivl_vs[p]

                @pl.when(r + 1 < RPW)
                def _prefetch():
                    pltpu.make_async_copy(
                        pos_hbm.at[b + 1], pos_vs[1 - p], sem_in[1 - p]).start()

                @pl.when(rp >= 1)
                def _drain():
                    pltpu.make_async_copy(cls_v, cls_hbm.at[b - 2], sem_cls[p]).wait()
                    pltpu.make_async_copy(w_v, w_hbm.at[b - 2], sem_w[p]).wait()
                    pltpu.make_async_copy(dl_v, delta_hbm.at[b - 2], sem_dl[p]).wait()

                rfw_b = plsc.load_gather(rfw_v, [jnp.full((_LN,), r, jnp.int32)])

                @pl.when(rp == 0)
                def _first_fill():
                    @plsc.parallel_loop(0, FW // _LN, unroll=8)
                    def _fill(c):
                        p0 = c * _LN
                        cls_v[pl.ds(p0, _LN)] = zeros_f
                        dval = jnp.float32(-0.5) - (p0 + iota_f)
                        dl_v[0, pl.ds(p0, _LN)] = dval
                        dl_v[1, pl.ds(p0, _LN)] = dval

                @pl.when(rp >= 1)
                def _undo():
                    @plsc.parallel_loop(0, L // _LN, unroll=4)
                    def _undo_loop(c):
                        l0 = c * _LN
                        old = ivl_v[pl.ds(l0, _LN)]
                        oldc = jnp.minimum(jnp.maximum(old, 0), FW - 1)
                        dval = jnp.float32(-0.5) - oldc.astype(jnp.float32)
                        plsc.store_scatter(cls_v, [oldc], zeros_f)
                        plsc.store_scatter(dl_v, [zeros_i, oldc], dval)
                        plsc.store_scatter(dl_v, [ones_i, oldc], dval)

                @plsc.parallel_loop(0, FW // _LN, unroll=8)
                def _wfill(c):
                    p0 = c * _LN
                    fvec = p0 + iota
                    w_v[pl.ds(p0, _LN)] = jnp.where(fvec <= rfw_b, _NEG_W, _PAD_W)

                pltpu.make_async_copy(pos_hbm.at[b], pos_v, sem_in[p]).wait()

                @plsc.parallel_loop(0, L // _LN, unroll=4)
                def _pass_a(c):
                    l0 = c * _LN
                    x1 = pos_v[0, pl.ds(l0, _LN)]
                    x2 = pos_v[1, pl.ds(l0, _LN)]
                    cs = (x1 + x2) * jnp.float32(0.5 / _STRIDE)
                    ivl = jnp.where(cs < 0.0, -1, cs.astype(jnp.int32))
                    ivl_v[pl.ds(l0, _LN)] = ivl

                @plsc.parallel_loop(0, L // _LN, unroll=4)
                def _pass_b(c):
                    l0 = c * _LN
                    li = l0 + iota
                    ivl = ivl_v[pl.ds(l0, _LN)]
                    prev = plsc.load_gather(ivl_v, [jnp.maximum(li - 1, 0)])
                    valid = ((ivl != prev) | (li == 0)) & (ivl >= 0) & (ivl < FW)
                    x1 = pos_v[0, pl.ds(l0, _LN)]
                    x2 = pos_v[1, pl.ds(l0, _LN)]
                    fvl = ivl.astype(jnp.float32) + 0.5
                    d0 = x1 * jnp.float32(1.0 / _STRIDE) - fvl
                    d1 = x2 * jnp.float32(1.0 / _STRIDE) - fvl
                    plsc.store_scatter(cls_v, [ivl], ones_f, mask=valid)
                    plsc.store_scatter(w_v, [ivl], pos_w, mask=valid)
                    plsc.store_scatter(dl_v, [zeros_i, ivl], d0, mask=valid)
                    plsc.store_scatter(dl_v, [ones_i, ivl], d1, mask=valid)

                pltpu.make_async_copy(cls_v, cls_hbm.at[b], sem_cls[p]).start()
                pltpu.make_async_copy(w_v, w_hbm.at[b], sem_w[p]).start()
                pltpu.make_async_copy(dl_v, delta_hbm.at[b], sem_dl[p]).start()
            return _
        lax.fori_loop(0, RPW // 2, pair_fn, None)

        for p in (0, 1):
            b_last = base + RPW - 2 + p
            pltpu.make_async_copy(cls_vs[p], cls_hbm.at[b_last], sem_cls[p]).wait()
            pltpu.make_async_copy(w_vs[p], w_hbm.at[b_last], sem_w[p]).wait()
            pltpu.make_async_copy(dl_vs[p], delta_hbm.at[b_last], sem_dl[p]).wait()

    return sc_call


def kernel(split_line_pos, feat_width, real_features_width):
    B, L, _ = split_line_pos.shape
    FW = 8192
    del feat_width
    rfw = real_features_width.astype(jnp.int32)
    pos_planes = split_line_pos.transpose(0, 2, 1)
    cls, w, delta_planes = _build_sc_call(B, L, FW)(pos_planes, rfw)
    return cls, w, delta_planes.transpose(0, 2, 1)
